# baseline (speedup 1.0000x reference)
import functools

import numpy as np
import jax
import jax.numpy as jnp
from jax import lax
from jax.experimental import pallas as pl
from jax.experimental.pallas import tpu as pltpu
from jax.experimental.pallas import tpu_sc as plsc

_OMEGA = 32
_SINK = 4
_KW = 3
_LOCAL_NUM = 4
_H = 16
_MAXC = 8192
_MAXW = (_MAXC - _SINK) // _OMEGA + 1
_S = 2048
_D = 128
_LOCAL = _LOCAL_NUM * _OMEGA
_SCORE_END = max(_SINK, _S - _LOCAL)
_NW = max(0, (_SCORE_END - _SINK) // _OMEGA)
_NWP = 64
_THR = _OMEGA / max(1.0, float(_S))
_KEEP = _SINK + _KW * _OMEGA + _LOCAL
_BQ = 512
_NQ = _S // _BQ
_HALF = 120
_IDXPAD = 2 * _HALF


def _window_sel() -> np.ndarray:
    c = np.arange(_S)[:, None]
    w = np.arange(_NWP)[None, :]
    sel = (w < _NW) & (c >= _SINK + _OMEGA * w) & (c < _SINK + _OMEGA * (w + 1))
    return sel.astype(np.float32)


def _tc_body(a0_ref, a1_ref, a2_ref, a3_ref, wsel_ref, e8_ref, votes_ref, stats_ref):
    h = pl.program_id(0)
    ones8 = e8_ref[...]

    def _half(tref):
        tile = tref[0]
        c8 = lax.dot_general(
            ones8, tile,
            (((1,), (0,)), ((), ())),
            precision=lax.Precision.HIGHEST,
            preferred_element_type=jnp.float32)
        win = lax.dot_general(
            tile, wsel_ref[...],
            (((1,), (0,)), ((), ())),
            precision=lax.Precision.DEFAULT,
            preferred_element_type=jnp.float32)
        hitf = (win > _THR).astype(jnp.float32)
        h8 = lax.dot_general(
            ones8, hitf,
            (((1,), (0,)), ((), ())),
            precision=lax.Precision.DEFAULT,
            preferred_element_type=jnp.float32)
        return c8, h8

    c8a, h8a = _half(a0_ref)
    c8b, h8b = _half(a1_ref)
    c8c, h8c = _half(a2_ref)
    c8d, h8d = _half(a3_ref)
    col8 = (c8a + c8b) + (c8c + c8d)
    ht8 = (h8a + h8b) + (h8c + h8d)
    colsum = col8[0:1, :]
    votes_ref[0] = jnp.zeros((1, _MAXC), jnp.float32)
    votes_ref[0, :, 0:_S] = colsum
    stats_ref[0] = ht8[0:1, :]


def _sel_body(st_ref, colsum_ref, wsel_ref, ws_ref, kept_ref):
    cum = lax.dot_general(
        colsum_ref[...], wsel_ref[...],
        (((1,), (0,)), ((), ())),
        precision=lax.Precision.HIGHEST,
        preferred_element_type=jnp.float32)
    hit = st_ref[...]
    lane = lax.broadcasted_iota(jnp.int32, (_H, _NWP), 1)
    neg = jnp.float32(-jnp.inf)
    c0 = jnp.where(lane < _NW, cum, neg)
    m0 = jnp.max(c0, axis=1, keepdims=True)
    a0 = jnp.min(jnp.where(c0 == m0, lane, _NWP), axis=1, keepdims=True)
    c1 = jnp.where(lane == a0, neg, c0)
    m1 = jnp.max(c1, axis=1, keepdims=True)
    a1 = jnp.min(jnp.where(c1 == m1, lane, _NWP), axis=1, keepdims=True)
    c2 = jnp.where(lane == a1, neg, c1)
    m2 = jnp.max(c2, axis=1, keepdims=True)
    a2 = jnp.min(jnp.where(c2 == m2, lane, _NWP), axis=1, keepdims=True)
    wa = jnp.minimum(a0, jnp.minimum(a1, a2))
    wc = jnp.maximum(a0, jnp.maximum(a1, a2))
    wb = a0 + a1 + a2 - wa - wc
    l = lax.broadcasted_iota(jnp.int32, (_H, _IDXPAD), 1)
    kept = jnp.where(
        l < _SINK, l,
        jnp.where(l < _SINK + _OMEGA, wa * _OMEGA + l,
                  jnp.where(l < _SINK + 2 * _OMEGA, wb * _OMEGA + l - _OMEGA,
                            jnp.where(l < _SINK + 3 * _OMEGA, wc * _OMEGA + l - 2 * _OMEGA,
                                      jnp.where(l < _KEEP, l + (_S - _LOCAL) - (_SINK + 3 * _OMEGA),
                                                _S - 1)))))
    hrow = lax.broadcasted_iota(jnp.int32, (_H, _IDXPAD), 0)
    kept_ref[...] = kept + hrow * _S
    lw = lax.broadcasted_iota(jnp.int32, (_H, _MAXW), 1)
    padn = jnp.full((_H, _MAXW - _NWP), jnp.nan, jnp.float32)
    nanv = jnp.full((_H, 1, _MAXW), jnp.nan, jnp.float32)
    cum_w = jnp.concatenate([cum, padn], axis=1)
    hit_w = jnp.concatenate([hit, padn], axis=1)
    row0 = jnp.where(lw < _NW, cum_w, jnp.nan)[:, None, :]
    row1 = jnp.where(lw < _NW, hit_w, jnp.nan)[:, None, :]
    ws_ref[...] = jnp.concatenate([row0, row1, nanv, nanv], axis=1)


_TC_KW = dict(
    grid=(_H,),
    in_specs=[
        pl.BlockSpec((1, _S // 4, _S), lambda h: (h, 0, 0)),
        pl.BlockSpec((1, _S // 4, _S), lambda h: (h, 1, 0)),
        pl.BlockSpec((1, _S // 4, _S), lambda h: (h, 2, 0)),
        pl.BlockSpec((1, _S // 4, _S), lambda h: (h, 3, 0)),
        pl.BlockSpec((_S, _NWP), lambda h: (0, 0)),
        pl.BlockSpec((8, _S // 4), lambda h: (0, 0)),
    ],
    out_specs=[
        pl.BlockSpec((1, 1, _MAXC), lambda h: (h, 0, 0)),
        pl.BlockSpec((1, 1, _NWP), lambda h: (h, 0, 0)),
    ],
    out_shape=[
        jax.ShapeDtypeStruct((_H, 1, _MAXC), jnp.float32),
        jax.ShapeDtypeStruct((_H, 1, _NWP), jnp.float32),
    ],
    compiler_params=pltpu.CompilerParams(
        dimension_semantics=("arbitrary",)),
)


def _sc_gather_call(key2, val2, kept):
    mesh = plsc.VectorSubcoreMesh(core_axis_name="c", subcore_axis_name="s")

    @functools.partial(
        pl.kernel,
        mesh=mesh,
        out_type=[
            jax.ShapeDtypeStruct((_H, _IDXPAD, _D), jnp.float32),
            jax.ShapeDtypeStruct((_H, _IDXPAD, _D), jnp.float32),
        ],
        scratch_types=[
            pltpu.VMEM((_HALF,), jnp.int32),
            pltpu.VMEM((_HALF, _D), jnp.float32),
            pltpu.SemaphoreType.DMA,
        ],
    )
    def sc_kernel(key_hbm, val_hbm, kept_hbm, ck_hbm, cv_hbm, idx_v, rows_v, sem):
        h = lax.axis_index("s")
        p = lax.axis_index("c")
        pltpu.sync_copy(kept_hbm.at[h, p], idx_v)
        cp1 = pltpu.async_copy(key_hbm.at[idx_v], rows_v, sem)
        cp1.wait()
        pltpu.sync_copy(rows_v, ck_hbm.at[h].at[pl.ds(p * _HALF, _HALF)])
        cp2 = pltpu.async_copy(val_hbm.at[idx_v], rows_v, sem)
        cp2.wait()
        pltpu.sync_copy(rows_v, cv_hbm.at[h].at[pl.ds(p * _HALF, _HALF)])

    return sc_kernel(key2, val2, kept)


def kernel(past_key, past_value, attn_score_cache):
    attn3 = attn_score_cache[0]
    wsel = jnp.asarray(_window_sel())
    e8 = jnp.zeros((8, _S // 4), jnp.float32).at[0].set(1.0)
    votes3, stats3 = pl.pallas_call(_tc_body, **_TC_KW)(attn3, attn3, attn3, attn3, wsel, e8)
    votes = votes3.reshape(_H, _MAXC)
    ws3, kept2 = pl.pallas_call(
        _sel_body,
        out_shape=[
            jax.ShapeDtypeStruct((_H, 4, _MAXW), jnp.float32),
            jax.ShapeDtypeStruct((_H, _IDXPAD), jnp.int32),
        ],
    )(stats3.reshape(_H, _NWP), votes3.reshape(_H, _MAXC)[:, 0:_S], wsel)
    window_scores = jnp.transpose(ws3, (0, 2, 1))
    key2 = past_key[0].reshape(_H * _S, _D)
    val2 = past_value[0].reshape(_H * _S, _D)
    kept = kept2.reshape(_H, 2, _HALF)
    ck_pad, cv_pad = _sc_gather_call(key2, val2, kept)
    return (ck_pad[None, :, :_KEEP], cv_pad[None, :, :_KEEP],
            window_scores, votes)


# speedup vs baseline: 1.1220x; 1.1220x over previous
import functools

import numpy as np
import jax
import jax.numpy as jnp
from jax import lax
from jax.experimental import pallas as pl
from jax.experimental.pallas import tpu as pltpu
from jax.experimental.pallas import tpu_sc as plsc

_OMEGA = 32
_SINK = 4
_KW = 3
_LOCAL_NUM = 4
_H = 16
_MAXC = 8192
_MAXW = (_MAXC - _SINK) // _OMEGA + 1
_S = 2048
_D = 128
_LOCAL = _LOCAL_NUM * _OMEGA
_SCORE_END = max(_SINK, _S - _LOCAL)
_NW = max(0, (_SCORE_END - _SINK) // _OMEGA)
_NWP = 64
_THR = _OMEGA / max(1.0, float(_S))
_KEEP = _SINK + _KW * _OMEGA + _LOCAL
_BQ = 512
_NQ = _S // _BQ
_HALF = 120
_IDXPAD = 2 * _HALF


def _window_sel() -> np.ndarray:
    c = np.arange(_S)[:, None]
    w = np.arange(_NWP)[None, :]
    sel = (w < _NW) & (c >= _SINK + _OMEGA * w) & (c < _SINK + _OMEGA * (w + 1))
    return sel.astype(np.float32)


def _tc_body(atop_ref, abot_ref, wsel_ref, e8_ref, votes_ref, stats_ref):
    h = pl.program_id(0)
    ones8 = e8_ref[...]

    def _half(tref):
        tile = tref[0]
        c8 = lax.dot_general(
            ones8, tile,
            (((1,), (0,)), ((), ())),
            precision=lax.Precision.HIGHEST,
            preferred_element_type=jnp.float32)
        win = lax.dot_general(
            tile, wsel_ref[...],
            (((1,), (0,)), ((), ())),
            precision=lax.Precision.DEFAULT,
            preferred_element_type=jnp.float32)
        hitf = (win > _THR).astype(jnp.float32)
        h8 = lax.dot_general(
            ones8, hitf,
            (((1,), (0,)), ((), ())),
            precision=lax.Precision.DEFAULT,
            preferred_element_type=jnp.float32)
        return c8, h8

    c8a, h8a = _half(atop_ref)
    c8b, h8b = _half(abot_ref)
    col8 = c8a + c8b
    ht8 = h8a + h8b
    colsum = col8[0:1, :]
    votes_ref[0] = jnp.zeros((1, _MAXC), jnp.float32)
    votes_ref[0, :, 0:_S] = colsum
    stats_ref[0] = ht8[0:1, :]


def _sel_body(st_ref, colsum_ref, wsel_ref, ws_ref, kept_ref):
    cum = lax.dot_general(
        colsum_ref[...], wsel_ref[...],
        (((1,), (0,)), ((), ())),
        precision=lax.Precision.HIGHEST,
        preferred_element_type=jnp.float32)
    hit = st_ref[...]
    lane = lax.broadcasted_iota(jnp.int32, (_H, _NWP), 1)
    neg = jnp.float32(-jnp.inf)
    c0 = jnp.where(lane < _NW, cum, neg)
    m0 = jnp.max(c0, axis=1, keepdims=True)
    a0 = jnp.min(jnp.where(c0 == m0, lane, _NWP), axis=1, keepdims=True)
    c1 = jnp.where(lane == a0, neg, c0)
    m1 = jnp.max(c1, axis=1, keepdims=True)
    a1 = jnp.min(jnp.where(c1 == m1, lane, _NWP), axis=1, keepdims=True)
    c2 = jnp.where(lane == a1, neg, c1)
    m2 = jnp.max(c2, axis=1, keepdims=True)
    a2 = jnp.min(jnp.where(c2 == m2, lane, _NWP), axis=1, keepdims=True)
    wa = jnp.minimum(a0, jnp.minimum(a1, a2))
    wc = jnp.maximum(a0, jnp.maximum(a1, a2))
    wb = a0 + a1 + a2 - wa - wc
    l = lax.broadcasted_iota(jnp.int32, (_H, _IDXPAD), 1)
    kept = jnp.where(
        l < _SINK, l,
        jnp.where(l < _SINK + _OMEGA, wa * _OMEGA + l,
                  jnp.where(l < _SINK + 2 * _OMEGA, wb * _OMEGA + l - _OMEGA,
                            jnp.where(l < _SINK + 3 * _OMEGA, wc * _OMEGA + l - 2 * _OMEGA,
                                      jnp.where(l < _KEEP, l + (_S - _LOCAL) - (_SINK + 3 * _OMEGA),
                                                _S - 1)))))
    hrow = lax.broadcasted_iota(jnp.int32, (_H, _IDXPAD), 0)
    kept_ref[...] = kept + hrow * _S
    lw = lax.broadcasted_iota(jnp.int32, (_H, _MAXW), 1)
    padn = jnp.full((_H, _MAXW - _NWP), jnp.nan, jnp.float32)
    nanv = jnp.full((_H, 1, _MAXW), jnp.nan, jnp.float32)
    cum_w = jnp.concatenate([cum, padn], axis=1)
    hit_w = jnp.concatenate([hit, padn], axis=1)
    row0 = jnp.where(lw < _NW, cum_w, jnp.nan)[:, None, :]
    row1 = jnp.where(lw < _NW, hit_w, jnp.nan)[:, None, :]
    ws_ref[...] = jnp.concatenate([row0, row1, nanv, nanv], axis=1)


_TC_KW = dict(
    grid=(_H,),
    in_specs=[
        pl.BlockSpec((1, _S // 2, _S), lambda h: (h, 0, 0)),
        pl.BlockSpec((1, _S // 2, _S), lambda h: (h, 1, 0)),
        pl.BlockSpec((_S, _NWP), lambda h: (0, 0)),
        pl.BlockSpec((8, _S // 2), lambda h: (0, 0)),
    ],
    out_specs=[
        pl.BlockSpec((1, 1, _MAXC), lambda h: (h, 0, 0)),
        pl.BlockSpec((1, 1, _NWP), lambda h: (h, 0, 0)),
    ],
    out_shape=[
        jax.ShapeDtypeStruct((_H, 1, _MAXC), jnp.float32),
        jax.ShapeDtypeStruct((_H, 1, _NWP), jnp.float32),
    ],
    compiler_params=pltpu.CompilerParams(
        dimension_semantics=("arbitrary",)),
)


def _sc_gather_call(key2, val2, kept):
    mesh = plsc.VectorSubcoreMesh(core_axis_name="c", subcore_axis_name="s")

    @functools.partial(
        pl.kernel,
        mesh=mesh,
        out_type=[
            jax.ShapeDtypeStruct((_H, _IDXPAD, _D), jnp.float32),
            jax.ShapeDtypeStruct((_H, _IDXPAD, _D), jnp.float32),
        ],
        scratch_types=[
            pltpu.VMEM((_HALF,), jnp.int32),
            pltpu.VMEM((_HALF, _D), jnp.float32),
            pltpu.SemaphoreType.DMA,
        ],
    )
    def sc_kernel(key_hbm, val_hbm, kept_hbm, ck_hbm, cv_hbm, idx_v, rows_v, sem):
        h = lax.axis_index("s")
        p = lax.axis_index("c")
        pltpu.sync_copy(kept_hbm.at[h, p], idx_v)
        cp1 = pltpu.async_copy(key_hbm.at[idx_v], rows_v, sem)
        cp1.wait()
        pltpu.sync_copy(rows_v, ck_hbm.at[h].at[pl.ds(p * _HALF, _HALF)])
        cp2 = pltpu.async_copy(val_hbm.at[idx_v], rows_v, sem)
        cp2.wait()
        pltpu.sync_copy(rows_v, cv_hbm.at[h].at[pl.ds(p * _HALF, _HALF)])

    return sc_kernel(key2, val2, kept)


def kernel(past_key, past_value, attn_score_cache):
    attn3 = attn_score_cache[0]
    wsel = jnp.asarray(_window_sel())
    e8 = jnp.zeros((8, _S // 2), jnp.float32).at[0].set(1.0)
    votes3, stats3 = pl.pallas_call(_tc_body, **_TC_KW)(attn3, attn3, wsel, e8)
    votes = votes3.reshape(_H, _MAXC)
    ws3, kept2 = pl.pallas_call(
        _sel_body,
        out_shape=[
            jax.ShapeDtypeStruct((_H, 4, _MAXW), jnp.float32),
            jax.ShapeDtypeStruct((_H, _IDXPAD), jnp.int32),
        ],
    )(stats3.reshape(_H, _NWP), votes3.reshape(_H, _MAXC)[:, 0:_S], wsel)
    window_scores = jnp.transpose(ws3, (0, 2, 1))
    key2 = past_key[0].reshape(_H * _S, _D)
    val2 = past_value[0].reshape(_H * _S, _D)
    kept = kept2.reshape(_H, 2, _HALF)
    ck_pad, cv_pad = _sc_gather_call(key2, val2, kept)
    return (ck_pad[None, :, :_KEEP], cv_pad[None, :, :_KEEP],
            window_scores, votes)
